# baseline (speedup 1.0000x reference)
import functools

import jax
import jax.numpy as jnp
from jax import lax
from jax.experimental import pallas as pl
from jax.experimental.pallas import tpu as pltpu
from jax.experimental.pallas import tpu_sc as plsc

NTOKEN = 1000000
NINP = 64
SEQ = 50
BATCH = 4096

NC = 2
NS = 16
NW = NC * NS
GW = BATCH // NW
NG = 5
NGROUP = SEQ // NG


def _body(table_hbm, inp_hbm, out_hbm, idx_v, rows0, rows1, gsem0, gsem1,
          wsem0, wsem1):
    wid = lax.axis_index("s") * NC + lax.axis_index("c")
    b0 = wid * GW

    pltpu.sync_copy(inp_hbm.at[:, pl.ds(b0, GW)], idx_v)

    def fire(g, rows, gsem):
        for j in range(NG):
            pltpu.async_copy(
                table_hbm.at[idx_v.at[g * NG + j]],
                rows.at[pl.ds(j * GW, GW)],
                gsem,
            )

    def drain_gathers(rows, gsem):
        pltpu.make_async_copy(
            table_hbm.at[pl.ds(0, NG * GW)], rows, gsem).wait()

    def write(g, rows, wsem):
        for j in range(NG):
            pltpu.async_copy(
                rows.at[pl.ds(j * GW, GW)],
                out_hbm.at[pl.ds(b0, GW), pl.ds((g * NG + j) * NINP, NINP)],
                wsem,
            )

    def drain_writes(rows, wsem):
        for j in range(NG):
            pltpu.make_async_copy(
                rows.at[pl.ds(j * GW, GW)],
                out_hbm.at[pl.ds(b0, GW), pl.ds(j * NINP, NINP)],
                wsem,
            ).wait()

    fire(0, rows0, gsem0)
    fire(1, rows1, gsem1)

    def loop_body(i, carry):
        s = i * 2
        drain_gathers(rows0, gsem0)
        write(s, rows0, wsem0)
        drain_gathers(rows1, gsem1)
        write(s + 1, rows1, wsem1)
        drain_writes(rows0, wsem0)
        fire(s + 2, rows0, gsem0)
        drain_writes(rows1, wsem1)
        fire(s + 3, rows1, gsem1)
        return carry

    lax.fori_loop(0, (NGROUP - 2) // 2, loop_body, 0)

    drain_gathers(rows0, gsem0)
    write(NGROUP - 2, rows0, wsem0)
    drain_gathers(rows1, gsem1)
    write(NGROUP - 1, rows1, wsem1)
    drain_writes(rows0, wsem0)
    drain_writes(rows1, wsem1)


_mesh = plsc.VectorSubcoreMesh(core_axis_name="c", subcore_axis_name="s")

_lookup = functools.partial(
    pl.kernel,
    mesh=_mesh,
    compiler_params=pltpu.CompilerParams(use_tc_tiling_on_sc=False),
    out_type=jax.ShapeDtypeStruct((BATCH, SEQ * NINP), jnp.bfloat16),
    scratch_types=[
        pltpu.VMEM((SEQ, GW), jnp.int32),
        pltpu.VMEM((NG * GW, NINP), jnp.bfloat16),
        pltpu.VMEM((NG * GW, NINP), jnp.bfloat16),
        pltpu.SemaphoreType.DMA,
        pltpu.SemaphoreType.DMA,
        pltpu.SemaphoreType.DMA,
        pltpu.SemaphoreType.DMA,
    ],
)(_body)


def kernel(inp, table, hidden):
    tbl = table.astype(jnp.bfloat16)
    return _lookup(tbl, inp).astype(jnp.float32)


# speedup vs baseline: 1.4042x; 1.4042x over previous
import functools

import jax
import jax.numpy as jnp
from jax import lax
from jax.experimental import pallas as pl
from jax.experimental.pallas import tpu as pltpu
from jax.experimental.pallas import tpu_sc as plsc

NTOKEN = 1000000
NINP = 64
SEQ = 50
BATCH = 4096

NC = 2
NS = 16
NW = NC * NS
GW = BATCH // NW
NG = 5
NGROUP = SEQ // NG


def _body(table_hbm, inp_hbm, out_hbm, idx_v, rows0, rows1, gsem0, gsem1,
          wsem0, wsem1):
    wid = lax.axis_index("s") * NC + lax.axis_index("c")
    b0 = wid * GW

    pltpu.sync_copy(inp_hbm.at[:, pl.ds(b0, GW)], idx_v)

    def fire(g, rows, gsem):
        for j in range(NG):
            pltpu.async_copy(
                table_hbm.at[idx_v.at[g * NG + j]],
                rows.at[pl.ds(j * GW, GW)],
                gsem,
            )

    def drain_gathers(rows, gsem):
        pltpu.make_async_copy(
            table_hbm.at[pl.ds(0, NG * GW)], rows, gsem).wait()

    def write(g, rows, wsem):
        for j in range(NG):
            pltpu.async_copy(
                rows.at[pl.ds(j * GW, GW)],
                out_hbm.at[pl.ds(b0, GW), pl.ds((g * NG + j) * NINP, NINP)],
                wsem,
            )

    def drain_writes(rows, wsem):
        for j in range(NG):
            pltpu.make_async_copy(
                rows.at[pl.ds(j * GW, GW)],
                out_hbm.at[pl.ds(b0, GW), pl.ds(j * NINP, NINP)],
                wsem,
            ).wait()

    fire(0, rows0, gsem0)
    fire(1, rows1, gsem1)

    def loop_body(i, carry):
        s = i * 2
        drain_gathers(rows0, gsem0)
        write(s, rows0, wsem0)
        drain_gathers(rows1, gsem1)
        write(s + 1, rows1, wsem1)
        drain_writes(rows0, wsem0)
        fire(s + 2, rows0, gsem0)
        drain_writes(rows1, wsem1)
        fire(s + 3, rows1, gsem1)
        return carry

    lax.fori_loop(0, (NGROUP - 2) // 2, loop_body, 0)

    drain_gathers(rows0, gsem0)
    write(NGROUP - 2, rows0, wsem0)
    drain_gathers(rows1, gsem1)
    write(NGROUP - 1, rows1, wsem1)
    drain_writes(rows0, wsem0)
    drain_writes(rows1, wsem1)


_mesh = plsc.VectorSubcoreMesh(core_axis_name="c", subcore_axis_name="s")

_lookup = functools.partial(
    pl.kernel,
    mesh=_mesh,
    compiler_params=pltpu.CompilerParams(use_tc_tiling_on_sc=False),
    out_type=jax.ShapeDtypeStruct((BATCH, SEQ * NINP), jnp.float32),
    scratch_types=[
        pltpu.VMEM((SEQ, GW), jnp.int32),
        pltpu.VMEM((NG * GW, NINP), jnp.float32),
        pltpu.VMEM((NG * GW, NINP), jnp.float32),
        pltpu.SemaphoreType.DMA,
        pltpu.SemaphoreType.DMA,
        pltpu.SemaphoreType.DMA,
        pltpu.SemaphoreType.DMA,
    ],
)(_body)


def kernel(inp, table, hidden):
    return _lookup(table, inp)
